# baseline (speedup 1.0000x reference)
import jax
import jax.numpy as jnp
from jax import lax
from jax.experimental import pallas as pl
from jax.experimental.pallas import tpu as pltpu
from jax.experimental.pallas import tpu_sc as plsc

N = 10000
E = 320000
D = 128
DH = D // 2
D_OUT = 40

NC = 2
NCU = 1
NS = 16
NW = NCU * NS

C = 128
K = 160
PH = 4
KP = K // PH
EPT = K * C
E_PAD = EPT * NW
NP = 10240
RPT = NP // NS

BM = 256


def _mesh():
    return plsc.VectorSubcoreMesh(core_axis_name="c", subcore_axis_name="s", num_cores=NCU)


def _degree_body(src_hbm, dst_hbm, degp_hbm, sidx, didx, hist_s, hist_d):
    c = lax.axis_index("c")
    s = lax.axis_index("s")
    wid = c * NS + s

    pltpu.sync_copy(src_hbm.at[wid], sidx)
    pltpu.sync_copy(dst_hbm.at[wid], didx)

    zero16 = jnp.zeros((16,), jnp.float32)

    def zinit(i, _):
        hist_s[pl.ds(i * 16, 16)] = zero16
        hist_d[pl.ds(i * 16, 16)] = zero16
        return 0

    lax.fori_loop(0, NP // 16, zinit, 0)

    ones16 = jnp.ones((16,), jnp.float32)

    def body(i, _):
        sv = sidx[pl.ds(i * 16, 16)]
        dv = didx[pl.ds(i * 16, 16)]
        plsc.addupdate_scatter(hist_s, [sv], ones16)
        plsc.addupdate_scatter(hist_d, [dv], ones16)
        return 0

    lax.fori_loop(0, (E_PAD // NW) // 16, body, 0)

    pltpu.sync_copy(hist_s, degp_hbm.at[0, wid])
    pltpu.sync_copy(hist_d, degp_hbm.at[1, wid])


@jax.jit
def _degrees(src_p, dst_p):
    return pl.kernel(
        _degree_body,
        out_type=jax.ShapeDtypeStruct((2, NW, NP), jnp.float32),
        mesh=_mesh(),
        compiler_params=pltpu.CompilerParams(needs_layout_passes=False),
        scratch_types=[
            pltpu.VMEM((E_PAD // NW,), jnp.int32),
            pltpu.VMEM((E_PAD // NW,), jnp.int32),
            pltpu.VMEM((NP,), jnp.float32),
            pltpu.VMEM((NP,), jnp.float32),
        ],
    )(src_p, dst_p)


def _agg_body(y_hbm, edge_hbm, out_hbm, eidx, rr, acc, gsem):
    c = lax.axis_index("c")
    s = lax.axis_index("s")
    wid = c * NS + s

    zero16 = jnp.zeros((16,), jnp.float32)

    def zrow(i, _):
        for k in range(D // 16):
            rr[0, i, pl.ds(k * 16, 16)] = zero16
        return 0

    lax.fori_loop(0, C, zrow, 0)

    def zspmem(k, _):
        pltpu.sync_copy(rr.at[0], acc.at[pl.ds(s * RPT + k * C, C)])
        return 0

    lax.fori_loop(0, RPT // C, zspmem, 0)

    plsc.subcore_barrier()

    K2 = KP
    for ph in range(PH):
        pltpu.sync_copy(edge_hbm.at[wid, ph], eidx)

        for b in range(2):
            pltpu.async_copy(y_hbm.at[eidx.at[0, b]], rr.at[b], gsem.at[b])

        def round_body(r, _):
            for b in range(2):
                j = 2 * r + b
                pltpu.make_async_copy(
                    y_hbm.at[eidx.at[0, j]], rr.at[b], gsem.at[b]).wait()
                pltpu.sync_copy(rr.at[b], acc.at[eidx.at[1, j]], add=True)
                pltpu.async_copy(y_hbm.at[eidx.at[0, j + 2]], rr.at[b],
                                 gsem.at[b])
            return 0

        lax.fori_loop(0, K2 // 2 - 1, round_body, 0)
        for b in range(2):
            j = K2 - 2 + b
            pltpu.make_async_copy(
                y_hbm.at[eidx.at[0, j]], rr.at[b], gsem.at[b]).wait()
            pltpu.sync_copy(rr.at[b], acc.at[eidx.at[1, j]], add=True)

    plsc.subcore_barrier()

    def copyout(k, _):
        pltpu.sync_copy(acc.at[pl.ds(s * RPT + k * C, C)], rr.at[0])
        pltpu.sync_copy(rr.at[0], out_hbm.at[c, pl.ds(s * RPT + k * C, C)])
        return 0

    lax.fori_loop(0, RPT // C, copyout, 0)


@jax.jit
def _aggregate(y_p, edge_p):
    return pl.kernel(
        _agg_body,
        out_type=jax.ShapeDtypeStruct((NCU, NP, D), jnp.float32),
        mesh=_mesh(),
        compiler_params=pltpu.CompilerParams(needs_layout_passes=False),
        scratch_types=[
            pltpu.VMEM((2, KP, C), jnp.int32),
            pltpu.VMEM((2, C, D), jnp.float32),
            pltpu.VMEM_SHARED((NP, D), jnp.float32),
            pltpu.SemaphoreType.DMA((2,)),
        ],
    )(y_p, edge_p)


def _norms(degb):
    deg_src = jnp.sum(degb[:NW], axis=0)
    deg_dst = jnp.sum(degb[NW:], axis=0)
    n_src = lax.rsqrt(jnp.clip(deg_src, 1.0, None))
    n_dst = lax.rsqrt(jnp.clip(deg_dst, 1.0, None))
    return n_src, n_dst


def _tca_body(xb, w1, degb, yb):
    n_src, _ = _norms(degb)
    yb[...] = jnp.dot(xb[...], w1[...],
                      preferred_element_type=jnp.float32) * n_src[:, None]


@jax.jit
def _tc_a(x_p, W1, degp):
    grid = NP // BM
    return pl.pallas_call(
        _tca_body,
        grid=(grid,),
        in_specs=[
            pl.BlockSpec((BM, D), lambda i: (i, 0)),
            pl.BlockSpec((D, D), lambda i: (0, 0)),
            pl.BlockSpec((2 * NW, BM), lambda i: (0, i)),
        ],
        out_specs=pl.BlockSpec((BM, D), lambda i: (i, 0)),
        out_shape=jax.ShapeDtypeStruct((NP, D), jnp.float32),
    )(x_p, W1, degp)


def _tcb_body(aggb, w2, b1b, degb, yb):
    n_src, n_dst = _norms(degb)
    row = pl.program_id(0) * BM + lax.broadcasted_iota(jnp.int32, (BM,), 0)
    n_src = jnp.where(row < N, n_src, 0.0)
    agg = sum(aggb[i] for i in range(1, NCU)) + aggb[0]
    h = jax.nn.relu(agg * n_dst[:, None] + b1b[...])
    yb[...] = jnp.dot(h, w2[...],
                      preferred_element_type=jnp.float32) * n_src[:, None]


@jax.jit
def _tc_b(agg, W2, b1, degp):
    grid = NP // BM
    return pl.pallas_call(
        _tcb_body,
        grid=(grid,),
        in_specs=[
            pl.BlockSpec((NCU, BM, D), lambda i: (0, i, 0)),
            pl.BlockSpec((D, D), lambda i: (0, 0)),
            pl.BlockSpec((1, D), lambda i: (0, 0)),
            pl.BlockSpec((2 * NW, BM), lambda i: (0, i)),
        ],
        out_specs=pl.BlockSpec((BM, D), lambda i: (i, 0)),
        out_shape=jax.ShapeDtypeStruct((NP, D), jnp.float32),
    )(agg, W2, b1.reshape(1, D), degp)


def _tcc_body(aggb, wc, b2b, bcb, degb, ob):
    _, n_dst = _norms(degb)
    agg = sum(aggb[i] for i in range(1, NCU)) + aggb[0]
    h = jax.nn.relu(agg * n_dst[:, None] + b2b[...])
    ob[...] = jnp.dot(h, wc[...], preferred_element_type=jnp.float32) + bcb[...]


@jax.jit
def _tc_c(agg, Wc_p, b2, bc_p, degp):
    grid = NP // BM
    return pl.pallas_call(
        _tcc_body,
        grid=(grid,),
        in_specs=[
            pl.BlockSpec((NCU, BM, D), lambda i: (0, i, 0)),
            pl.BlockSpec((D, D), lambda i: (0, 0)),
            pl.BlockSpec((1, D), lambda i: (0, 0)),
            pl.BlockSpec((1, D), lambda i: (0, 0)),
            pl.BlockSpec((2 * NW, BM), lambda i: (0, i)),
        ],
        out_specs=pl.BlockSpec((BM, D), lambda i: (i, 0)),
        out_shape=jax.ShapeDtypeStruct((NP, D), jnp.float32),
    )(agg, Wc_p, b2.reshape(1, D), bc_p, degp)


def kernel(x, edge_index, W1, b1, W2, b2, Wc, bc):
    src = edge_index[0].astype(jnp.int32)
    dst = edge_index[1].astype(jnp.int32)
    padfill = jnp.full((E_PAD - E,), N, jnp.int32)
    src_p = jnp.concatenate([src, padfill])
    dst_p = jnp.concatenate([dst, padfill])
    edge_p = jnp.stack([src_p.reshape(NW, K, C),
                        dst_p.reshape(NW, K, C)], axis=1)
    edge_p = edge_p.reshape(NW, 2, PH, KP, C).transpose(0, 2, 1, 3, 4)
    src_d = src_p.reshape(NW, E_PAD // NW)
    dst_d = dst_p.reshape(NW, E_PAD // NW)

    x_p = jnp.pad(x, ((0, NP - N), (0, 0)))
    Wc_p = jnp.pad(Wc, ((0, 0), (0, D - D_OUT)))
    bc_p = jnp.pad(bc, ((0, D - D_OUT),)).reshape(1, D)

    degp = _degrees(src_d, dst_d).reshape(2 * NW, NP)

    y1 = _tc_a(x_p, W1, degp)
    agg1 = _aggregate(y1, edge_p)
    y2 = _tc_b(agg1, W2, b1, degp)
    agg2 = _aggregate(y2, edge_p)
    out = _tc_c(agg2, Wc_p, b2, bc_p, degp)
    return out[:N, :D_OUT]


# speedup vs baseline: 1.2110x; 1.2110x over previous
import jax
import jax.numpy as jnp
from jax import lax
from jax.experimental import pallas as pl
from jax.experimental.pallas import tpu as pltpu
from jax.experimental.pallas import tpu_sc as plsc

N = 10000
E = 320000
D = 128
DH = D // 2
D_OUT = 40

NC = 2
NCU = 2
NS = 16
NW = NCU * NS

CE = 128
KS = 8
NSL0 = 15
NSL1 = 5
K0 = NSL0 * KS
K1 = NSL1 * KS
E_PAD = NS * (K0 + K1) * CE
NP = 10240
RPT = NP // NS

BM = 256


def _mesh():
    return plsc.VectorSubcoreMesh(core_axis_name="c", subcore_axis_name="s", num_cores=NCU)


def _degree_body(src_hbm, dst_hbm, degp_hbm, sidx, didx, hist_s, hist_d):
    c = lax.axis_index("c")
    s = lax.axis_index("s")
    wid = c * NS + s

    pltpu.sync_copy(src_hbm.at[wid], sidx)
    pltpu.sync_copy(dst_hbm.at[wid], didx)

    zero16 = jnp.zeros((16,), jnp.float32)

    def zinit(i, _):
        hist_s[pl.ds(i * 16, 16)] = zero16
        hist_d[pl.ds(i * 16, 16)] = zero16
        return 0

    lax.fori_loop(0, NP // 16, zinit, 0)

    ones16 = jnp.ones((16,), jnp.float32)

    def body(i, _):
        sv = sidx[pl.ds(i * 16, 16)]
        dv = didx[pl.ds(i * 16, 16)]
        plsc.addupdate_scatter(hist_s, [sv], ones16)
        plsc.addupdate_scatter(hist_d, [dv], ones16)
        return 0

    lax.fori_loop(0, (E_PAD // NW) // 16, body, 0)

    pltpu.sync_copy(hist_s, degp_hbm.at[0, wid])
    pltpu.sync_copy(hist_d, degp_hbm.at[1, wid])


@jax.jit
def _degrees(src_p, dst_p):
    return pl.kernel(
        _degree_body,
        out_type=jax.ShapeDtypeStruct((2, NW, NP), jnp.float32),
        mesh=_mesh(),
        compiler_params=pltpu.CompilerParams(needs_layout_passes=False),
        scratch_types=[
            pltpu.VMEM((E_PAD // NW,), jnp.int32),
            pltpu.VMEM((E_PAD // NW,), jnp.int32),
            pltpu.VMEM((NP,), jnp.float32),
            pltpu.VMEM((NP,), jnp.float32),
        ],
    )(src_p, dst_p)


def _agg_body(y_hbm, edge_hbm, out_hbm, ib0, ib1, rr, acc, gsem, isem):
    c = lax.axis_index("c")
    s = lax.axis_index("s")
    wid = c * NS + s

    zero16 = jnp.zeros((16,), jnp.float32)

    def zrow(i, _):
        for k in range(D // 16):
            rr[0, i, pl.ds(k * 16, 16)] = zero16
        return 0

    lax.fori_loop(0, CE, zrow, 0)

    def zspmem(k, _):
        pltpu.sync_copy(rr.at[0], acc.at[pl.ds(s * RPT + k * CE, CE)])
        return 0

    lax.fori_loop(0, RPT // CE, zspmem, 0)

    plsc.subcore_barrier()

    nslab = lax.select(c == 0, NSL0, NSL1)
    npairs = (nslab - 1) // 2
    last = nslab - 1

    def process(ib):
        hs = {}
        for b in range(2):
            hs[b] = pltpu.async_copy(y_hbm.at[ib.at[0, b]], rr.at[b],
                                     gsem.at[b])
        for j in range(KS):
            b = j % 2
            hs[j].wait()
            pltpu.sync_copy(rr.at[b], acc.at[ib.at[1, j]], add=True)
            if j + 2 < KS:
                hs[j + 2] = pltpu.async_copy(y_hbm.at[ib.at[0, j + 2]],
                                             rr.at[b], gsem.at[b])

    pltpu.sync_copy(edge_hbm.at[wid, 0], ib0)
    pltpu.async_copy(edge_hbm.at[wid, 1], ib1, isem.at[1])

    def pair_body(q, _):
        process(ib0)
        pltpu.async_copy(
            edge_hbm.at[wid, jnp.minimum(2 * q + 2, last)], ib0, isem.at[0])
        pltpu.make_async_copy(edge_hbm.at[wid, 0], ib1, isem.at[1]).wait()
        process(ib1)
        pltpu.async_copy(
            edge_hbm.at[wid, jnp.minimum(2 * q + 3, last)], ib1, isem.at[1])
        pltpu.make_async_copy(edge_hbm.at[wid, 0], ib0, isem.at[0]).wait()
        return 0

    lax.fori_loop(0, npairs, pair_body, 0)
    pltpu.make_async_copy(edge_hbm.at[wid, 0], ib1, isem.at[1]).wait()
    process(ib0)

    plsc.subcore_barrier()

    def copyout(k, _):
        pltpu.sync_copy(acc.at[pl.ds(s * RPT + k * CE, CE)], rr.at[0])
        pltpu.sync_copy(rr.at[0], out_hbm.at[c, pl.ds(s * RPT + k * CE, CE)])
        return 0

    lax.fori_loop(0, RPT // CE, copyout, 0)


@jax.jit
def _aggregate(y_p, edge_p):
    return pl.kernel(
        _agg_body,
        out_type=jax.ShapeDtypeStruct((NCU, NP, D), jnp.float32),
        mesh=_mesh(),
        compiler_params=pltpu.CompilerParams(needs_layout_passes=False),
        scratch_types=[
            pltpu.VMEM((2, KS, CE), jnp.int32),
            pltpu.VMEM((2, KS, CE), jnp.int32),
            pltpu.VMEM((2, CE, D), jnp.float32),
            pltpu.VMEM_SHARED((NP, D), jnp.float32),
            pltpu.SemaphoreType.DMA((2,)),
            pltpu.SemaphoreType.DMA((2,)),
        ],
    )(y_p, edge_p)


def _norms(degb):
    deg_src = jnp.sum(degb[:NW], axis=0)
    deg_dst = jnp.sum(degb[NW:], axis=0)
    n_src = lax.rsqrt(jnp.clip(deg_src, 1.0, None))
    n_dst = lax.rsqrt(jnp.clip(deg_dst, 1.0, None))
    return n_src, n_dst


def _tca_body(xb, w1, degb, yb):
    n_src, _ = _norms(degb)
    yb[...] = jnp.dot(xb[...], w1[...],
                      preferred_element_type=jnp.float32) * n_src[:, None]


@jax.jit
def _tc_a(x_p, W1, degp):
    grid = NP // BM
    return pl.pallas_call(
        _tca_body,
        grid=(grid,),
        in_specs=[
            pl.BlockSpec((BM, D), lambda i: (i, 0)),
            pl.BlockSpec((D, D), lambda i: (0, 0)),
            pl.BlockSpec((2 * NW, BM), lambda i: (0, i)),
        ],
        out_specs=pl.BlockSpec((BM, D), lambda i: (i, 0)),
        out_shape=jax.ShapeDtypeStruct((NP, D), jnp.float32),
    )(x_p, W1, degp)


def _tcb_body(aggb, w2, b1b, degb, yb):
    n_src, n_dst = _norms(degb)
    row = pl.program_id(0) * BM + lax.broadcasted_iota(jnp.int32, (BM,), 0)
    n_src = jnp.where(row < N, n_src, 0.0)
    agg = sum(aggb[i] for i in range(1, NCU)) + aggb[0]
    h = jax.nn.relu(agg * n_dst[:, None] + b1b[...])
    yb[...] = jnp.dot(h, w2[...],
                      preferred_element_type=jnp.float32) * n_src[:, None]


@jax.jit
def _tc_b(agg, W2, b1, degp):
    grid = NP // BM
    return pl.pallas_call(
        _tcb_body,
        grid=(grid,),
        in_specs=[
            pl.BlockSpec((NCU, BM, D), lambda i: (0, i, 0)),
            pl.BlockSpec((D, D), lambda i: (0, 0)),
            pl.BlockSpec((1, D), lambda i: (0, 0)),
            pl.BlockSpec((2 * NW, BM), lambda i: (0, i)),
        ],
        out_specs=pl.BlockSpec((BM, D), lambda i: (i, 0)),
        out_shape=jax.ShapeDtypeStruct((NP, D), jnp.float32),
    )(agg, W2, b1.reshape(1, D), degp)


def _tcc_body(aggb, wc, b2b, bcb, degb, ob):
    _, n_dst = _norms(degb)
    agg = sum(aggb[i] for i in range(1, NCU)) + aggb[0]
    h = jax.nn.relu(agg * n_dst[:, None] + b2b[...])
    ob[...] = jnp.dot(h, wc[...], preferred_element_type=jnp.float32) + bcb[...]


@jax.jit
def _tc_c(agg, Wc_p, b2, bc_p, degp):
    grid = NP // BM
    return pl.pallas_call(
        _tcc_body,
        grid=(grid,),
        in_specs=[
            pl.BlockSpec((NCU, BM, D), lambda i: (0, i, 0)),
            pl.BlockSpec((D, D), lambda i: (0, 0)),
            pl.BlockSpec((1, D), lambda i: (0, 0)),
            pl.BlockSpec((1, D), lambda i: (0, 0)),
            pl.BlockSpec((2 * NW, BM), lambda i: (0, i)),
        ],
        out_specs=pl.BlockSpec((BM, D), lambda i: (i, 0)),
        out_shape=jax.ShapeDtypeStruct((NP, D), jnp.float32),
    )(agg, Wc_p, b2.reshape(1, D), bc_p, degp)


def kernel(x, edge_index, W1, b1, W2, b2, Wc, bc):
    src = edge_index[0].astype(jnp.int32)
    dst = edge_index[1].astype(jnp.int32)
    padfill = jnp.full((E_PAD - E,), N, jnp.int32)
    src_p = jnp.concatenate([src, padfill])
    dst_p = jnp.concatenate([dst, padfill])

    e0 = NS * K0 * CE

    def _tile_view(a):
        a0 = a[:e0].reshape(NS, NSL0, KS, CE)
        a1 = a[e0:].reshape(NS, NSL1, KS, CE)
        a1 = jnp.pad(a1, ((0, 0), (0, NSL0 - NSL1), (0, 0), (0, 0)),
                     constant_values=N)
        return jnp.concatenate([a0, a1], axis=0)

    edge_p = jnp.stack([_tile_view(src_p), _tile_view(dst_p)], axis=2)
    src_d = src_p.reshape(NW, E_PAD // NW)
    dst_d = dst_p.reshape(NW, E_PAD // NW)

    x_p = jnp.pad(x, ((0, NP - N), (0, 0)))
    Wc_p = jnp.pad(Wc, ((0, 0), (0, D - D_OUT)))
    bc_p = jnp.pad(bc, ((0, D - D_OUT),)).reshape(1, D)

    degp = _degrees(src_d, dst_d).reshape(2 * NW, NP)

    y1 = _tc_a(x_p, W1, degp)
    agg1 = _aggregate(y1, edge_p)
    y2 = _tc_b(agg1, W2, b1, degp)
    agg2 = _aggregate(y2, edge_p)
    out = _tc_c(agg2, Wc_p, b2, bc_p, degp)
    return out[:N, :D_OUT]


# speedup vs baseline: 1.2170x; 1.0049x over previous
import jax
import jax.numpy as jnp
from jax import lax
from jax.experimental import pallas as pl
from jax.experimental.pallas import tpu as pltpu
from jax.experimental.pallas import tpu_sc as plsc

N = 10000
E = 320000
D = 128
DH = D // 2
D_OUT = 40

NC = 2
NCU = 2
NS = 16
NW = NCU * NS

CE = 128
KP = 40
NPH0 = 3
NPH1 = 1
K0 = NPH0 * KP
K1 = NPH1 * KP
E_PAD = NS * (K0 + K1) * CE
NP = 10240
RPT = NP // NS

BM = 256


def _mesh():
    return plsc.VectorSubcoreMesh(core_axis_name="c", subcore_axis_name="s", num_cores=NCU)


def _degree_body(src_hbm, dst_hbm, degp_hbm, sidx, didx, hist_s, hist_d):
    c = lax.axis_index("c")
    s = lax.axis_index("s")
    wid = c * NS + s

    pltpu.sync_copy(src_hbm.at[wid], sidx)
    pltpu.sync_copy(dst_hbm.at[wid], didx)

    zero16 = jnp.zeros((16,), jnp.float32)

    def zinit(i, _):
        hist_s[pl.ds(i * 16, 16)] = zero16
        hist_d[pl.ds(i * 16, 16)] = zero16
        return 0

    lax.fori_loop(0, NP // 16, zinit, 0)

    ones16 = jnp.ones((16,), jnp.float32)

    def body(i, _):
        sv = sidx[pl.ds(i * 16, 16)]
        dv = didx[pl.ds(i * 16, 16)]
        plsc.addupdate_scatter(hist_s, [sv], ones16)
        plsc.addupdate_scatter(hist_d, [dv], ones16)
        return 0

    lax.fori_loop(0, (E_PAD // NW) // 16, body, 0)

    pltpu.sync_copy(hist_s, degp_hbm.at[0, wid])
    pltpu.sync_copy(hist_d, degp_hbm.at[1, wid])


@jax.jit
def _degrees(src_p, dst_p):
    return pl.kernel(
        _degree_body,
        out_type=jax.ShapeDtypeStruct((2, NW, NP), jnp.float32),
        mesh=_mesh(),
        compiler_params=pltpu.CompilerParams(needs_layout_passes=False),
        scratch_types=[
            pltpu.VMEM((E_PAD // NW,), jnp.int32),
            pltpu.VMEM((E_PAD // NW,), jnp.int32),
            pltpu.VMEM((NP,), jnp.float32),
            pltpu.VMEM((NP,), jnp.float32),
        ],
    )(src_p, dst_p)


def _agg_body(y_hbm, edge_hbm, out_hbm, eidx, rr, acc, gsem):
    c = lax.axis_index("c")
    s = lax.axis_index("s")
    wid = c * NS + s

    zero16 = jnp.zeros((16,), jnp.float32)

    def zrow(i, _):
        for k in range(D // 16):
            rr[0, i, pl.ds(k * 16, 16)] = zero16
        return 0

    lax.fori_loop(0, CE, zrow, 0)

    def zspmem(k, _):
        pltpu.sync_copy(rr.at[0], acc.at[pl.ds(s * RPT + k * CE, CE)])
        return 0

    lax.fori_loop(0, RPT // CE, zspmem, 0)

    plsc.subcore_barrier()

    nph = lax.select(c == 0, NPH0, NPH1)

    def phase_body(ph, _):
        pltpu.sync_copy(edge_hbm.at[wid, ph], eidx)

        for b in range(2):
            pltpu.async_copy(y_hbm.at[eidx.at[0, b]], rr.at[b], gsem.at[b])

        def round_body(r, _):
            for b in range(2):
                j = 2 * r + b
                pltpu.make_async_copy(
                    y_hbm.at[eidx.at[0, j]], rr.at[b], gsem.at[b]).wait()
                pltpu.sync_copy(rr.at[b], acc.at[eidx.at[1, j]], add=True)
                pltpu.async_copy(y_hbm.at[eidx.at[0, j + 2]], rr.at[b],
                                 gsem.at[b])
            return 0

        lax.fori_loop(0, KP // 2 - 1, round_body, 0)
        for b in range(2):
            j = KP - 2 + b
            pltpu.make_async_copy(
                y_hbm.at[eidx.at[0, j]], rr.at[b], gsem.at[b]).wait()
            pltpu.sync_copy(rr.at[b], acc.at[eidx.at[1, j]], add=True)
        return 0

    lax.fori_loop(0, nph, phase_body, 0)

    plsc.subcore_barrier()

    def copyout(k, _):
        pltpu.sync_copy(acc.at[pl.ds(s * RPT + k * CE, CE)], rr.at[0])
        pltpu.sync_copy(rr.at[0], out_hbm.at[c, pl.ds(s * RPT + k * CE, CE)])
        return 0

    lax.fori_loop(0, RPT // CE, copyout, 0)


@jax.jit
def _aggregate(y_p, edge_p):
    return pl.kernel(
        _agg_body,
        out_type=jax.ShapeDtypeStruct((NCU, NP, D), jnp.float32),
        mesh=_mesh(),
        compiler_params=pltpu.CompilerParams(needs_layout_passes=False),
        scratch_types=[
            pltpu.VMEM((2, KP, CE), jnp.int32),
            pltpu.VMEM((2, CE, D), jnp.float32),
            pltpu.VMEM_SHARED((NP, D), jnp.float32),
            pltpu.SemaphoreType.DMA((2,)),
        ],
    )(y_p, edge_p)


def _norms(degb):
    deg_src = jnp.sum(degb[:NW], axis=0)
    deg_dst = jnp.sum(degb[NW:], axis=0)
    n_src = lax.rsqrt(jnp.clip(deg_src, 1.0, None))
    n_dst = lax.rsqrt(jnp.clip(deg_dst, 1.0, None))
    return n_src, n_dst


def _tca_body(xb, w1, degb, yb):
    n_src, _ = _norms(degb)
    yb[...] = jnp.dot(xb[...], w1[...],
                      preferred_element_type=jnp.float32) * n_src[:, None]


@jax.jit
def _tc_a(x_p, W1, degp):
    grid = NP // BM
    return pl.pallas_call(
        _tca_body,
        grid=(grid,),
        in_specs=[
            pl.BlockSpec((BM, D), lambda i: (i, 0)),
            pl.BlockSpec((D, D), lambda i: (0, 0)),
            pl.BlockSpec((2 * NW, BM), lambda i: (0, i)),
        ],
        out_specs=pl.BlockSpec((BM, D), lambda i: (i, 0)),
        out_shape=jax.ShapeDtypeStruct((NP, D), jnp.float32),
    )(x_p, W1, degp)


def _tcb_body(aggb, w2, b1b, degb, yb):
    n_src, n_dst = _norms(degb)
    row = pl.program_id(0) * BM + lax.broadcasted_iota(jnp.int32, (BM,), 0)
    n_src = jnp.where(row < N, n_src, 0.0)
    agg = sum(aggb[i] for i in range(1, NCU)) + aggb[0]
    h = jax.nn.relu(agg * n_dst[:, None] + b1b[...])
    yb[...] = jnp.dot(h, w2[...],
                      preferred_element_type=jnp.float32) * n_src[:, None]


@jax.jit
def _tc_b(agg, W2, b1, degp):
    grid = NP // BM
    return pl.pallas_call(
        _tcb_body,
        grid=(grid,),
        in_specs=[
            pl.BlockSpec((NCU, BM, D), lambda i: (0, i, 0)),
            pl.BlockSpec((D, D), lambda i: (0, 0)),
            pl.BlockSpec((1, D), lambda i: (0, 0)),
            pl.BlockSpec((2 * NW, BM), lambda i: (0, i)),
        ],
        out_specs=pl.BlockSpec((BM, D), lambda i: (i, 0)),
        out_shape=jax.ShapeDtypeStruct((NP, D), jnp.float32),
    )(agg, W2, b1.reshape(1, D), degp)


def _tcc_body(aggb, wc, b2b, bcb, degb, ob):
    _, n_dst = _norms(degb)
    agg = sum(aggb[i] for i in range(1, NCU)) + aggb[0]
    h = jax.nn.relu(agg * n_dst[:, None] + b2b[...])
    ob[...] = jnp.dot(h, wc[...], preferred_element_type=jnp.float32) + bcb[...]


@jax.jit
def _tc_c(agg, Wc_p, b2, bc_p, degp):
    grid = NP // BM
    return pl.pallas_call(
        _tcc_body,
        grid=(grid,),
        in_specs=[
            pl.BlockSpec((NCU, BM, D), lambda i: (0, i, 0)),
            pl.BlockSpec((D, D), lambda i: (0, 0)),
            pl.BlockSpec((1, D), lambda i: (0, 0)),
            pl.BlockSpec((1, D), lambda i: (0, 0)),
            pl.BlockSpec((2 * NW, BM), lambda i: (0, i)),
        ],
        out_specs=pl.BlockSpec((BM, D), lambda i: (i, 0)),
        out_shape=jax.ShapeDtypeStruct((NP, D), jnp.float32),
    )(agg, Wc_p, b2.reshape(1, D), bc_p, degp)


def kernel(x, edge_index, W1, b1, W2, b2, Wc, bc):
    src = edge_index[0].astype(jnp.int32)
    dst = edge_index[1].astype(jnp.int32)
    padfill = jnp.full((E_PAD - E,), N, jnp.int32)
    src_p = jnp.concatenate([src, padfill])
    dst_p = jnp.concatenate([dst, padfill])

    e0 = NS * K0 * CE

    def _tile_view(a):
        a0 = a[:e0].reshape(NS, NPH0, KP, CE)
        a1 = a[e0:].reshape(NS, NPH1, KP, CE)
        a1 = jnp.pad(a1, ((0, 0), (0, NPH0 - NPH1), (0, 0), (0, 0)),
                     constant_values=N)
        return jnp.concatenate([a0, a1], axis=0)

    edge_p = jnp.stack([_tile_view(src_p), _tile_view(dst_p)], axis=2)
    src_d = src_p.reshape(NW, E_PAD // NW)
    dst_d = dst_p.reshape(NW, E_PAD // NW)

    x_p = jnp.pad(x, ((0, NP - N), (0, 0)))
    Wc_p = jnp.pad(Wc, ((0, 0), (0, D - D_OUT)))
    bc_p = jnp.pad(bc, ((0, D - D_OUT),)).reshape(1, D)

    degp = _degrees(src_d, dst_d).reshape(2 * NW, NP)

    y1 = _tc_a(x_p, W1, degp)
    agg1 = _aggregate(y1, edge_p)
    y2 = _tc_b(agg1, W2, b1, degp)
    agg2 = _aggregate(y2, edge_p)
    out = _tc_c(agg2, Wc_p, b2, bc_p, degp)
    return out[:N, :D_OUT]


# speedup vs baseline: 1.2919x; 1.0615x over previous
import jax
import jax.numpy as jnp
from jax import lax
from jax.experimental import pallas as pl
from jax.experimental.pallas import tpu as pltpu
from jax.experimental.pallas import tpu_sc as plsc

N = 10000
E = 320000
D = 128
DH = D // 2
D_OUT = 40

NC = 2
NCU = 2
NS = 16
NW = NCU * NS

CE = 128
KP = 40
NPH0 = 3
NPH1 = 1
K0 = NPH0 * KP
K1 = NPH1 * KP
E_PAD = NS * (K0 + K1) * CE
NP = 10240
RPT = NP // NS

BM = 256


def _mesh():
    return plsc.VectorSubcoreMesh(core_axis_name="c", subcore_axis_name="s", num_cores=NCU)


def _degree_body(src_hbm, dst_hbm, degp_hbm, sidx, didx, hist_s, hist_d):
    c = lax.axis_index("c")
    s = lax.axis_index("s")
    wid = c * NS + s

    pltpu.sync_copy(src_hbm.at[wid], sidx)
    pltpu.sync_copy(dst_hbm.at[wid], didx)

    zero16 = jnp.zeros((16,), jnp.float32)

    def zinit(i, _):
        hist_s[pl.ds(i * 16, 16)] = zero16
        hist_d[pl.ds(i * 16, 16)] = zero16
        return 0

    lax.fori_loop(0, NP // 16, zinit, 0)

    ones16 = jnp.ones((16,), jnp.float32)

    def body(i, _):
        sv = sidx[pl.ds(i * 16, 16)]
        dv = didx[pl.ds(i * 16, 16)]
        plsc.addupdate_scatter(hist_s, [sv], ones16)
        plsc.addupdate_scatter(hist_d, [dv], ones16)
        return 0

    lax.fori_loop(0, (E_PAD // NW) // 16, body, 0)

    pltpu.sync_copy(hist_s, degp_hbm.at[0, wid])
    pltpu.sync_copy(hist_d, degp_hbm.at[1, wid])


@jax.jit
def _degrees(src_p, dst_p):
    return pl.kernel(
        _degree_body,
        out_type=jax.ShapeDtypeStruct((2, NW, NP), jnp.float32),
        mesh=_mesh(),
        compiler_params=pltpu.CompilerParams(needs_layout_passes=False),
        scratch_types=[
            pltpu.VMEM((E_PAD // NW,), jnp.int32),
            pltpu.VMEM((E_PAD // NW,), jnp.int32),
            pltpu.VMEM((NP,), jnp.float32),
            pltpu.VMEM((NP,), jnp.float32),
        ],
    )(src_p, dst_p)


def _agg_body(y_hbm, edge_hbm, out_hbm, eidx, rr, acc, gsem):
    c = lax.axis_index("c")
    s = lax.axis_index("s")
    wid = c * NS + s

    zero16 = jnp.zeros((16,), jnp.float32)

    def zrow(i, _):
        for k in range(D // 16):
            rr[0, i, pl.ds(k * 16, 16)] = zero16
        return 0

    lax.fori_loop(0, CE, zrow, 0)

    def zspmem(k, _):
        pltpu.sync_copy(rr.at[0], acc.at[pl.ds(s * RPT + k * CE, CE)])
        return 0

    lax.fori_loop(0, RPT // CE, zspmem, 0)

    plsc.subcore_barrier()

    nph = lax.select(c == 0, NPH1, NPH0)

    def phase_body(ph, _):
        pltpu.sync_copy(edge_hbm.at[wid, ph], eidx)

        for b in range(2):
            pltpu.async_copy(y_hbm.at[eidx.at[0, b]], rr.at[b], gsem.at[b])

        def round_body(r, _):
            for b in range(2):
                j = 2 * r + b
                pltpu.make_async_copy(
                    y_hbm.at[eidx.at[0, j]], rr.at[b], gsem.at[b]).wait()
                pltpu.sync_copy(rr.at[b], acc.at[eidx.at[1, j]], add=True)
                pltpu.async_copy(y_hbm.at[eidx.at[0, j + 2]], rr.at[b],
                                 gsem.at[b])
            return 0

        lax.fori_loop(0, KP // 2 - 1, round_body, 0)
        for b in range(2):
            j = KP - 2 + b
            pltpu.make_async_copy(
                y_hbm.at[eidx.at[0, j]], rr.at[b], gsem.at[b]).wait()
            pltpu.sync_copy(rr.at[b], acc.at[eidx.at[1, j]], add=True)
        return 0

    lax.fori_loop(0, nph, phase_body, 0)

    plsc.subcore_barrier()

    def copyout(k, _):
        pltpu.sync_copy(acc.at[pl.ds(s * RPT + k * CE, CE)], rr.at[0])
        pltpu.sync_copy(rr.at[0], out_hbm.at[c, pl.ds(s * RPT + k * CE, CE)])
        return 0

    lax.fori_loop(0, RPT // CE, copyout, 0)


@jax.jit
def _aggregate(y_p, edge_p):
    return pl.kernel(
        _agg_body,
        out_type=jax.ShapeDtypeStruct((NCU, NP, D), jnp.float32),
        mesh=_mesh(),
        compiler_params=pltpu.CompilerParams(needs_layout_passes=False),
        scratch_types=[
            pltpu.VMEM((2, KP, CE), jnp.int32),
            pltpu.VMEM((2, CE, D), jnp.float32),
            pltpu.VMEM_SHARED((NP, D), jnp.float32),
            pltpu.SemaphoreType.DMA((2,)),
        ],
    )(y_p, edge_p)


def _norms(degb):
    deg_src = jnp.sum(degb[:NW], axis=0)
    deg_dst = jnp.sum(degb[NW:], axis=0)
    n_src = lax.rsqrt(jnp.clip(deg_src, 1.0, None))
    n_dst = lax.rsqrt(jnp.clip(deg_dst, 1.0, None))
    return n_src, n_dst


def _tca_body(xb, w1, degb, yb):
    n_src, _ = _norms(degb)
    yb[...] = jnp.dot(xb[...], w1[...],
                      preferred_element_type=jnp.float32) * n_src[:, None]


@jax.jit
def _tc_a(x_p, W1, degp):
    grid = NP // BM
    return pl.pallas_call(
        _tca_body,
        grid=(grid,),
        in_specs=[
            pl.BlockSpec((BM, D), lambda i: (i, 0)),
            pl.BlockSpec((D, D), lambda i: (0, 0)),
            pl.BlockSpec((2 * NW, BM), lambda i: (0, i)),
        ],
        out_specs=pl.BlockSpec((BM, D), lambda i: (i, 0)),
        out_shape=jax.ShapeDtypeStruct((NP, D), jnp.float32),
    )(x_p, W1, degp)


def _tcb_body(aggb, w2, b1b, degb, yb):
    n_src, n_dst = _norms(degb)
    row = pl.program_id(0) * BM + lax.broadcasted_iota(jnp.int32, (BM,), 0)
    n_src = jnp.where(row < N, n_src, 0.0)
    agg = sum(aggb[i] for i in range(1, NCU)) + aggb[0]
    h = jax.nn.relu(agg * n_dst[:, None] + b1b[...])
    yb[...] = jnp.dot(h, w2[...],
                      preferred_element_type=jnp.float32) * n_src[:, None]


@jax.jit
def _tc_b(agg, W2, b1, degp):
    grid = NP // BM
    return pl.pallas_call(
        _tcb_body,
        grid=(grid,),
        in_specs=[
            pl.BlockSpec((NCU, BM, D), lambda i: (0, i, 0)),
            pl.BlockSpec((D, D), lambda i: (0, 0)),
            pl.BlockSpec((1, D), lambda i: (0, 0)),
            pl.BlockSpec((2 * NW, BM), lambda i: (0, i)),
        ],
        out_specs=pl.BlockSpec((BM, D), lambda i: (i, 0)),
        out_shape=jax.ShapeDtypeStruct((NP, D), jnp.float32),
    )(agg, W2, b1.reshape(1, D), degp)


def _tcc_body(aggb, wc, b2b, bcb, degb, ob):
    _, n_dst = _norms(degb)
    agg = sum(aggb[i] for i in range(1, NCU)) + aggb[0]
    h = jax.nn.relu(agg * n_dst[:, None] + b2b[...])
    ob[...] = jnp.dot(h, wc[...], preferred_element_type=jnp.float32) + bcb[...]


@jax.jit
def _tc_c(agg, Wc_p, b2, bc_p, degp):
    grid = NP // BM
    return pl.pallas_call(
        _tcc_body,
        grid=(grid,),
        in_specs=[
            pl.BlockSpec((NCU, BM, D), lambda i: (0, i, 0)),
            pl.BlockSpec((D, D), lambda i: (0, 0)),
            pl.BlockSpec((1, D), lambda i: (0, 0)),
            pl.BlockSpec((1, D), lambda i: (0, 0)),
            pl.BlockSpec((2 * NW, BM), lambda i: (0, i)),
        ],
        out_specs=pl.BlockSpec((BM, D), lambda i: (i, 0)),
        out_shape=jax.ShapeDtypeStruct((NP, D), jnp.float32),
    )(agg, Wc_p, b2.reshape(1, D), bc_p, degp)


def kernel(x, edge_index, W1, b1, W2, b2, Wc, bc):
    src = edge_index[0].astype(jnp.int32)
    dst = edge_index[1].astype(jnp.int32)
    padfill = jnp.full((E_PAD - E,), N, jnp.int32)
    src_p = jnp.concatenate([src, padfill])
    dst_p = jnp.concatenate([dst, padfill])

    e0 = NS * K0 * CE

    def _tile_view(a):
        a0 = a[:e0].reshape(NS, NPH0, KP, CE)
        a1 = a[e0:].reshape(NS, NPH1, KP, CE)
        a1 = jnp.pad(a1, ((0, 0), (0, NPH0 - NPH1), (0, 0), (0, 0)),
                     constant_values=N)
        return jnp.concatenate([a1, a0], axis=0)

    edge_p = jnp.stack([_tile_view(src_p), _tile_view(dst_p)], axis=2)
    src_d = src_p.reshape(NW, E_PAD // NW)
    dst_d = dst_p.reshape(NW, E_PAD // NW)

    x_p = jnp.pad(x, ((0, NP - N), (0, 0)))
    Wc_p = jnp.pad(Wc, ((0, 0), (0, D - D_OUT)))
    bc_p = jnp.pad(bc, ((0, D - D_OUT),)).reshape(1, D)

    degp = _degrees(src_d, dst_d).reshape(2 * NW, NP)

    y1 = _tc_a(x_p, W1, degp)
    agg1 = _aggregate(y1, edge_p)
    y2 = _tc_b(agg1, W2, b1, degp)
    agg2 = _aggregate(y2, edge_p)
    out = _tc_c(agg2, Wc_p, b2, bc_p, degp)
    return out[:N, :D_OUT]


# speedup vs baseline: 1.5552x; 1.2038x over previous
import jax
import jax.numpy as jnp
from jax import lax
from jax.experimental import pallas as pl
from jax.experimental.pallas import tpu as pltpu
from jax.experimental.pallas import tpu_sc as plsc

N = 10000
E = 320000
D = 128
DH = D // 2
D_OUT = 40

NC = 2
NCU = 2
NS = 16
NW = NCU * NS

CE = 128
KP = 40
NPH0 = 2
NPH1 = 2
K0 = NPH0 * KP
K1 = NPH1 * KP
E_PAD = NS * (K0 + K1) * CE
NP = 10240
RPT = NP // NS

BM = 256


def _mesh():
    return plsc.VectorSubcoreMesh(core_axis_name="c", subcore_axis_name="s", num_cores=NCU)


def _degree_body(src_hbm, dst_hbm, degp_hbm, sidx, didx, hist_s, hist_d):
    c = lax.axis_index("c")
    s = lax.axis_index("s")
    wid = c * NS + s

    pltpu.sync_copy(src_hbm.at[wid], sidx)
    pltpu.sync_copy(dst_hbm.at[wid], didx)

    zero16 = jnp.zeros((16,), jnp.float32)

    def zinit(i, _):
        hist_s[pl.ds(i * 16, 16)] = zero16
        hist_d[pl.ds(i * 16, 16)] = zero16
        return 0

    lax.fori_loop(0, NP // 16, zinit, 0)

    ones16 = jnp.ones((16,), jnp.float32)

    def body(i, _):
        sv = sidx[pl.ds(i * 16, 16)]
        dv = didx[pl.ds(i * 16, 16)]
        plsc.addupdate_scatter(hist_s, [sv], ones16)
        plsc.addupdate_scatter(hist_d, [dv], ones16)
        return 0

    lax.fori_loop(0, (E_PAD // NW) // 16, body, 0)

    pltpu.sync_copy(hist_s, degp_hbm.at[0, wid])
    pltpu.sync_copy(hist_d, degp_hbm.at[1, wid])


@jax.jit
def _degrees(src_p, dst_p):
    return pl.kernel(
        _degree_body,
        out_type=jax.ShapeDtypeStruct((2, NW, NP), jnp.float32),
        mesh=_mesh(),
        compiler_params=pltpu.CompilerParams(needs_layout_passes=False),
        scratch_types=[
            pltpu.VMEM((E_PAD // NW,), jnp.int32),
            pltpu.VMEM((E_PAD // NW,), jnp.int32),
            pltpu.VMEM((NP,), jnp.float32),
            pltpu.VMEM((NP,), jnp.float32),
        ],
    )(src_p, dst_p)


def _agg_body(y_hbm, edge_hbm, zeros_hbm, out_hbm, eidx, rr, acc, gsem):
    c = lax.axis_index("c")
    s = lax.axis_index("s")
    wid = c * NS + s

    pltpu.sync_copy(zeros_hbm.at[pl.ds(s * RPT, RPT)],
                    acc.at[pl.ds(s * RPT, RPT)])

    plsc.subcore_barrier()

    nph = NPH0

    def phase_body(ph, _):
        pltpu.sync_copy(edge_hbm.at[wid, ph], eidx)

        for b in range(2):
            pltpu.async_copy(y_hbm.at[eidx.at[0, b]], rr.at[b], gsem.at[b])

        def round_body(r, _):
            for b in range(2):
                j = 2 * r + b
                pltpu.make_async_copy(
                    y_hbm.at[eidx.at[0, j]], rr.at[b], gsem.at[b]).wait()
                pltpu.sync_copy(rr.at[b], acc.at[eidx.at[1, j]], add=True)
                pltpu.async_copy(y_hbm.at[eidx.at[0, j + 2]], rr.at[b],
                                 gsem.at[b])
            return 0

        lax.fori_loop(0, KP // 2 - 1, round_body, 0)
        for b in range(2):
            j = KP - 2 + b
            pltpu.make_async_copy(
                y_hbm.at[eidx.at[0, j]], rr.at[b], gsem.at[b]).wait()
            pltpu.sync_copy(rr.at[b], acc.at[eidx.at[1, j]], add=True)
        return 0

    lax.fori_loop(0, nph, phase_body, 0)

    plsc.subcore_barrier()

    pltpu.sync_copy(acc.at[pl.ds(s * RPT, RPT)],
                    out_hbm.at[c, pl.ds(s * RPT, RPT)])


@jax.jit
def _aggregate(y_p, edge_p, zeros_p):
    return pl.kernel(
        _agg_body,
        out_type=jax.ShapeDtypeStruct((NCU, NP, D), jnp.float32),
        mesh=_mesh(),
        compiler_params=pltpu.CompilerParams(needs_layout_passes=False),
        scratch_types=[
            pltpu.VMEM((2, KP, CE), jnp.int32),
            pltpu.VMEM((2, CE, D), jnp.float32),
            pltpu.VMEM_SHARED((NP, D), jnp.float32),
            pltpu.SemaphoreType.DMA((2,)),
        ],
    )(y_p, edge_p, zeros_p)


def _norms(degb):
    deg_src = jnp.sum(degb[:NW], axis=0)
    deg_dst = jnp.sum(degb[NW:], axis=0)
    n_src = lax.rsqrt(jnp.clip(deg_src, 1.0, None))
    n_dst = lax.rsqrt(jnp.clip(deg_dst, 1.0, None))
    return n_src, n_dst


def _tca_body(xb, w1, degb, yb):
    n_src, _ = _norms(degb)
    yb[...] = jnp.dot(xb[...], w1[...],
                      preferred_element_type=jnp.float32) * n_src[:, None]


@jax.jit
def _tc_a(x_p, W1, degp):
    grid = NP // BM
    return pl.pallas_call(
        _tca_body,
        grid=(grid,),
        in_specs=[
            pl.BlockSpec((BM, D), lambda i: (i, 0)),
            pl.BlockSpec((D, D), lambda i: (0, 0)),
            pl.BlockSpec((2 * NW, BM), lambda i: (0, i)),
        ],
        out_specs=pl.BlockSpec((BM, D), lambda i: (i, 0)),
        out_shape=jax.ShapeDtypeStruct((NP, D), jnp.float32),
    )(x_p, W1, degp)


def _tcb_body(aggb, w2, b1b, degb, yb):
    n_src, n_dst = _norms(degb)
    row = pl.program_id(0) * BM + lax.broadcasted_iota(jnp.int32, (BM,), 0)
    n_src = jnp.where(row < N, n_src, 0.0)
    agg = sum(aggb[i] for i in range(1, NCU)) + aggb[0]
    h = jax.nn.relu(agg * n_dst[:, None] + b1b[...])
    yb[...] = jnp.dot(h, w2[...],
                      preferred_element_type=jnp.float32) * n_src[:, None]


@jax.jit
def _tc_b(agg, W2, b1, degp):
    grid = NP // BM
    return pl.pallas_call(
        _tcb_body,
        grid=(grid,),
        in_specs=[
            pl.BlockSpec((NCU, BM, D), lambda i: (0, i, 0)),
            pl.BlockSpec((D, D), lambda i: (0, 0)),
            pl.BlockSpec((1, D), lambda i: (0, 0)),
            pl.BlockSpec((2 * NW, BM), lambda i: (0, i)),
        ],
        out_specs=pl.BlockSpec((BM, D), lambda i: (i, 0)),
        out_shape=jax.ShapeDtypeStruct((NP, D), jnp.float32),
    )(agg, W2, b1.reshape(1, D), degp)


def _tcc_body(aggb, wc, b2b, bcb, degb, ob):
    _, n_dst = _norms(degb)
    agg = sum(aggb[i] for i in range(1, NCU)) + aggb[0]
    h = jax.nn.relu(agg * n_dst[:, None] + b2b[...])
    ob[...] = jnp.dot(h, wc[...], preferred_element_type=jnp.float32) + bcb[...]


@jax.jit
def _tc_c(agg, Wc_p, b2, bc_p, degp):
    grid = NP // BM
    return pl.pallas_call(
        _tcc_body,
        grid=(grid,),
        in_specs=[
            pl.BlockSpec((NCU, BM, D), lambda i: (0, i, 0)),
            pl.BlockSpec((D, D), lambda i: (0, 0)),
            pl.BlockSpec((1, D), lambda i: (0, 0)),
            pl.BlockSpec((1, D), lambda i: (0, 0)),
            pl.BlockSpec((2 * NW, BM), lambda i: (0, i)),
        ],
        out_specs=pl.BlockSpec((BM, D), lambda i: (i, 0)),
        out_shape=jax.ShapeDtypeStruct((NP, D), jnp.float32),
    )(agg, Wc_p, b2.reshape(1, D), bc_p, degp)


def kernel(x, edge_index, W1, b1, W2, b2, Wc, bc):
    src = edge_index[0].astype(jnp.int32)
    dst = edge_index[1].astype(jnp.int32)
    padfill = jnp.full((E_PAD - E,), N, jnp.int32)
    src_p = jnp.concatenate([src, padfill])
    dst_p = jnp.concatenate([dst, padfill])

    e0 = NS * K0 * CE

    def _tile_view(a):
        a0 = a[:e0].reshape(NS, NPH0, KP, CE)
        a1 = a[e0:].reshape(NS, NPH1, KP, CE)
        a1 = jnp.pad(a1, ((0, 0), (0, NPH0 - NPH1), (0, 0), (0, 0)),
                     constant_values=N)
        return jnp.concatenate([a1, a0], axis=0)

    edge_p = jnp.stack([_tile_view(src_p), _tile_view(dst_p)], axis=2)
    zeros_p = jnp.zeros((NP, D), jnp.float32)
    src_d = src_p.reshape(NW, E_PAD // NW)
    dst_d = dst_p.reshape(NW, E_PAD // NW)

    x_p = jnp.pad(x, ((0, NP - N), (0, 0)))
    Wc_p = jnp.pad(Wc, ((0, 0), (0, D - D_OUT)))
    bc_p = jnp.pad(bc, ((0, D - D_OUT),)).reshape(1, D)

    degp = _degrees(src_d, dst_d).reshape(2 * NW, NP)

    y1 = _tc_a(x_p, W1, degp)
    agg1 = _aggregate(y1, edge_p, zeros_p)
    y2 = _tc_b(agg1, W2, b1, degp)
    agg2 = _aggregate(y2, edge_p, zeros_p)
    out = _tc_c(agg2, Wc_p, b2, bc_p, degp)
    return out[:N, :D_OUT]
